# Initial kernel scaffold; baseline (speedup 1.0000x reference)
import functools

import jax
import jax.numpy as jnp
from jax.experimental import pallas as pl

_HI = jax.lax.Precision.HIGHEST


def _top3_weights(q, c, T, S):
    qn = jnp.sum(q * q, axis=0)
    cn = jnp.sum(c * c, axis=0)
    cross = jax.lax.dot_general(q, c, (((0,), (0,)), ((), ())), precision=_HI)
    d = (qn[:, None] - 2.0 * cross) + cn[None, :]
    inf = jnp.float32(jnp.inf)
    m1 = jnp.min(d, axis=1)
    d1 = jnp.where(d == m1[:, None], inf, d)
    m2 = jnp.min(d1, axis=1)
    d2 = jnp.where(d1 == m2[:, None], inf, d1)
    m3 = jnp.min(d2, axis=1)
    r1 = 1.0 / (m1 + 1e-8)
    r2 = 1.0 / (m2 + 1e-8)
    r3 = 1.0 / (m3 + 1e-8)
    inv_norm = 1.0 / (r1 + r2 + r3)
    W = jnp.where(d <= m3[:, None],
                  (1.0 / (d + 1e-8)) * inv_norm[:, None],
                  jnp.float32(0.0))
    return W


def _stage_a(xyz1_ref, xyz2_ref, p1_ref, p2_ref, w0_ref, b0_ref,
             x1_ref, st_ref, *, T, S):
    q = xyz1_ref[0]
    feats = [p1_ref[0]]
    for l in (1, 0):
        W = _top3_weights(q, xyz2_ref[l, 0], T, S)
        pts = p2_ref[l, 0]
        interp_t = jax.lax.dot_general(
            pts, W, (((1,), (1,)), ((), ())), precision=_HI)
        feats.append(interp_t)
    feat_t = jnp.concatenate(feats, axis=0)
    x1 = jax.lax.dot_general(
        w0_ref[...], feat_t, (((1,), (0,)), ((), ())), precision=_HI)
    x1 = x1 + b0_ref[...]
    x1_ref[0] = x1
    first = (pl.program_id(0) == 0) & (pl.program_id(1) == 0)

    @pl.when(first)
    def _():
        st_ref[...] = jnp.zeros_like(st_ref)

    s = jnp.sum(x1, axis=1, keepdims=True)
    sq = jnp.sum(x1 * x1, axis=1, keepdims=True)
    st_ref[...] += jnp.concatenate([s, sq], axis=1)


def _stage_b(x1_ref, st1_ref, g0_ref, bt0_ref, w1_ref, b1_ref,
             x2_ref, st_ref, *, count):
    st = st1_ref[...]
    mean = st[:, 0:1] * (1.0 / count)
    var = st[:, 1:2] * (1.0 / count) - mean * mean
    rstd = jax.lax.rsqrt(var + 1e-5)
    h = (x1_ref[0] - mean) * (rstd * g0_ref[...]) + bt0_ref[...]
    h = jnp.maximum(h, 0.0)
    x2 = jax.lax.dot_general(
        w1_ref[...], h, (((1,), (0,)), ((), ())), precision=_HI)
    x2 = x2 + b1_ref[...]
    x2_ref[0] = x2
    first = (pl.program_id(0) == 0) & (pl.program_id(1) == 0)

    @pl.when(first)
    def _():
        st_ref[...] = jnp.zeros_like(st_ref)

    s = jnp.sum(x2, axis=1, keepdims=True)
    sq = jnp.sum(x2 * x2, axis=1, keepdims=True)
    st_ref[...] += jnp.concatenate([s, sq], axis=1)


def _stage_c(x2_ref, st2_ref, g1_ref, bt1_ref, out_ref, *, count):
    st = st2_ref[...]
    mean = st[:, 0:1] * (1.0 / count)
    var = st[:, 1:2] * (1.0 / count) - mean * mean
    rstd = jax.lax.rsqrt(var + 1e-5)
    y = (x2_ref[0] - mean) * (rstd * g1_ref[...]) + bt1_ref[...]
    out_ref[0] = jnp.maximum(y, 0.0)


def kernel(xyz1, xyz2_list, points1, points2_list,
           conv_w0, conv_b0, gamma0, beta0,
           conv_w1, conv_b1, gamma1, beta1):
    B, _, N = xyz1.shape
    L, _, _, S = xyz2_list.shape
    D1 = points1.shape[1]
    D2 = points2_list.shape[2]
    C1 = conv_w0.shape[0]
    C2 = conv_w1.shape[0]
    T = 512 if N % 512 == 0 else N
    NT = N // T
    count = float(B * N)

    col = lambda v: v.reshape(-1, 1)
    b0, g0, bt0 = col(conv_b0), col(gamma0), col(beta0)
    b1, g1, bt1 = col(conv_b1), col(gamma1), col(beta1)

    grid = (B, NT)
    full2 = lambda b, n: (0, 0)

    x1, st1 = pl.pallas_call(
        functools.partial(_stage_a, T=T, S=S),
        grid=grid,
        in_specs=[
            pl.BlockSpec((1, 3, T), lambda b, n: (b, 0, n)),
            pl.BlockSpec((L, 1, 3, S), lambda b, n: (0, b, 0, 0)),
            pl.BlockSpec((1, D1, T), lambda b, n: (b, 0, n)),
            pl.BlockSpec((L, 1, D2, S), lambda b, n: (0, b, 0, 0)),
            pl.BlockSpec((C1, D1 + L * D2), full2),
            pl.BlockSpec((C1, 1), full2),
        ],
        out_specs=[
            pl.BlockSpec((1, C1, T), lambda b, n: (b, 0, n)),
            pl.BlockSpec((C1, 2), full2),
        ],
        out_shape=[
            jax.ShapeDtypeStruct((B, C1, N), jnp.float32),
            jax.ShapeDtypeStruct((C1, 2), jnp.float32),
        ],
    )(xyz1, xyz2_list, points1, points2_list, conv_w0, b0)

    x2, st2 = pl.pallas_call(
        functools.partial(_stage_b, count=count),
        grid=grid,
        in_specs=[
            pl.BlockSpec((1, C1, T), lambda b, n: (b, 0, n)),
            pl.BlockSpec((C1, 2), full2),
            pl.BlockSpec((C1, 1), full2),
            pl.BlockSpec((C1, 1), full2),
            pl.BlockSpec((C2, C1), full2),
            pl.BlockSpec((C2, 1), full2),
        ],
        out_specs=[
            pl.BlockSpec((1, C2, T), lambda b, n: (b, 0, n)),
            pl.BlockSpec((C2, 2), full2),
        ],
        out_shape=[
            jax.ShapeDtypeStruct((B, C2, N), jnp.float32),
            jax.ShapeDtypeStruct((C2, 2), jnp.float32),
        ],
    )(x1, st1, g0, bt0, conv_w1, b1)

    out = pl.pallas_call(
        functools.partial(_stage_c, count=count),
        grid=grid,
        in_specs=[
            pl.BlockSpec((1, C2, T), lambda b, n: (b, 0, n)),
            pl.BlockSpec((C2, 2), full2),
            pl.BlockSpec((C2, 1), full2),
            pl.BlockSpec((C2, 1), full2),
        ],
        out_specs=pl.BlockSpec((1, C2, T), lambda b, n: (b, 0, n)),
        out_shape=jax.ShapeDtypeStruct((B, C2, N), jnp.float32),
    )(x2, st2, g1, bt1)

    return out


# speedup vs baseline: 16.0295x; 16.0295x over previous
import functools

import jax
import jax.numpy as jnp
from jax.experimental import pallas as pl

_HI = jax.lax.Precision.HIGHEST


def _top3_weights(q, c, T, S):
    qn = jnp.sum(q * q, axis=1)
    cn = jnp.sum(c * c, axis=1)
    cross = jax.lax.dot_general(
        q.astype(jnp.bfloat16), c.astype(jnp.bfloat16),
        (((1,), (1,)), ((), ())),
        preferred_element_type=jnp.float32)
    d = (qn[:, None] - 2.0 * cross) + cn[None, :]
    inf = jnp.float32(jnp.inf)
    one = jnp.float32(1.0)
    zero = jnp.float32(0.0)
    m1 = jnp.min(d, axis=1)
    oh1 = jnp.where(d == m1[:, None], one, zero)
    d1 = jnp.where(d == m1[:, None], inf, d)
    m2 = jnp.min(d1, axis=1)
    oh2 = jnp.where(d1 == m2[:, None], one, zero)
    d2 = jnp.where(d1 == m2[:, None], inf, d1)
    m3 = jnp.min(d2, axis=1)
    oh3 = jnp.where(d2 == m3[:, None], one, zero)
    r1 = 1.0 / (m1 + 1e-8)
    r2 = 1.0 / (m2 + 1e-8)
    r3 = 1.0 / (m3 + 1e-8)
    norm = (r1 + r2) + r3
    return (oh1, oh2, oh3), (r1 / norm, r2 / norm, r3 / norm)


def _stage_a(xyz1_ref, xyz2_ref, p1_ref, p2_ref, w0_ref, b0_ref,
             x1_ref, st_ref, *, T, S):
    q = xyz1_ref[0]
    feats = [p1_ref[0]]
    for l in (1, 0):
        ohs, ws = _top3_weights(q, xyz2_ref[l, 0], T, S)
        pts = p2_ref[l, 0]
        gs = [jax.lax.dot_general(pts, oh, (((1,), (1,)), ((), ())),
                                  precision=_HI) for oh in ohs]
        interp_t = ((gs[0] * ws[0][None, :] + gs[1] * ws[1][None, :])
                    + gs[2] * ws[2][None, :])
        feats.append(interp_t)
    feat_t = jnp.concatenate(feats, axis=0)
    x1 = jax.lax.dot_general(
        w0_ref[...].astype(jnp.bfloat16), feat_t.astype(jnp.bfloat16),
        (((1,), (0,)), ((), ())), preferred_element_type=jnp.float32)
    x1 = x1 + b0_ref[...]
    x1_ref[0] = x1
    first = (pl.program_id(0) == 0) & (pl.program_id(1) == 0)

    @pl.when(first)
    def _():
        st_ref[...] = jnp.zeros_like(st_ref)

    s = jnp.sum(x1, axis=1, keepdims=True)
    sq = jnp.sum(x1 * x1, axis=1, keepdims=True)
    st_ref[...] += jnp.concatenate([s, sq], axis=1)


def _stage_b(x1_ref, st1_ref, g0_ref, bt0_ref, w1_ref, b1_ref,
             x2_ref, st_ref, *, count):
    st = st1_ref[...]
    mean = st[:, 0:1] * (1.0 / count)
    var = st[:, 1:2] * (1.0 / count) - mean * mean
    rstd = 1.0 / jnp.sqrt(var + 1e-5)
    h = (x1_ref[0] - mean) * (rstd * g0_ref[...]) + bt0_ref[...]
    h = jnp.maximum(h, 0.0)
    x2 = jax.lax.dot_general(
        w1_ref[...].astype(jnp.bfloat16), h.astype(jnp.bfloat16),
        (((1,), (0,)), ((), ())), preferred_element_type=jnp.float32)
    x2 = x2 + b1_ref[...]
    x2_ref[0] = x2
    first = (pl.program_id(0) == 0) & (pl.program_id(1) == 0)

    @pl.when(first)
    def _():
        st_ref[...] = jnp.zeros_like(st_ref)

    s = jnp.sum(x2, axis=1, keepdims=True)
    sq = jnp.sum(x2 * x2, axis=1, keepdims=True)
    st_ref[...] += jnp.concatenate([s, sq], axis=1)


def _stage_c(x2_ref, st2_ref, g1_ref, bt1_ref, out_ref, *, count):
    st = st2_ref[...]
    mean = st[:, 0:1] * (1.0 / count)
    var = st[:, 1:2] * (1.0 / count) - mean * mean
    rstd = 1.0 / jnp.sqrt(var + 1e-5)
    y = (x2_ref[0] - mean) * (rstd * g1_ref[...]) + bt1_ref[...]
    out_ref[0] = jnp.maximum(y, 0.0)


def kernel(xyz1, xyz2_list, points1, points2_list,
           conv_w0, conv_b0, gamma0, beta0,
           conv_w1, conv_b1, gamma1, beta1):
    B, _, N = xyz1.shape
    L, _, _, S = xyz2_list.shape
    D1 = points1.shape[1]
    D2 = points2_list.shape[2]
    C1 = conv_w0.shape[0]
    C2 = conv_w1.shape[0]
    T = 512 if N % 512 == 0 else N
    NT = N // T
    count = float(B * N)

    col = lambda v: v.reshape(-1, 1)
    b0, g0, bt0 = col(conv_b0), col(gamma0), col(beta0)
    b1, g1, bt1 = col(conv_b1), col(gamma1), col(beta1)
    xyz1_t = jnp.transpose(xyz1, (0, 2, 1))
    xyz2_t = jnp.transpose(xyz2_list, (0, 1, 3, 2))

    grid = (B, NT)
    full2 = lambda b, n: (0, 0)

    x1, st1 = pl.pallas_call(
        functools.partial(_stage_a, T=T, S=S),
        grid=grid,
        in_specs=[
            pl.BlockSpec((1, T, 3), lambda b, n: (b, n, 0)),
            pl.BlockSpec((L, 1, S, 3), lambda b, n: (0, b, 0, 0)),
            pl.BlockSpec((1, D1, T), lambda b, n: (b, 0, n)),
            pl.BlockSpec((L, 1, D2, S), lambda b, n: (0, b, 0, 0)),
            pl.BlockSpec((C1, D1 + L * D2), full2),
            pl.BlockSpec((C1, 1), full2),
        ],
        out_specs=[
            pl.BlockSpec((1, C1, T), lambda b, n: (b, 0, n)),
            pl.BlockSpec((C1, 2), full2),
        ],
        out_shape=[
            jax.ShapeDtypeStruct((B, C1, N), jnp.float32),
            jax.ShapeDtypeStruct((C1, 2), jnp.float32),
        ],
    )(xyz1_t, xyz2_t, points1, points2_list, conv_w0, b0)

    x2, st2 = pl.pallas_call(
        functools.partial(_stage_b, count=count),
        grid=grid,
        in_specs=[
            pl.BlockSpec((1, C1, T), lambda b, n: (b, 0, n)),
            pl.BlockSpec((C1, 2), full2),
            pl.BlockSpec((C1, 1), full2),
            pl.BlockSpec((C1, 1), full2),
            pl.BlockSpec((C2, C1), full2),
            pl.BlockSpec((C2, 1), full2),
        ],
        out_specs=[
            pl.BlockSpec((1, C2, T), lambda b, n: (b, 0, n)),
            pl.BlockSpec((C2, 2), full2),
        ],
        out_shape=[
            jax.ShapeDtypeStruct((B, C2, N), jnp.float32),
            jax.ShapeDtypeStruct((C2, 2), jnp.float32),
        ],
    )(x1, st1, g0, bt0, conv_w1, b1)

    out = pl.pallas_call(
        functools.partial(_stage_c, count=count),
        grid=grid,
        in_specs=[
            pl.BlockSpec((1, C2, T), lambda b, n: (b, 0, n)),
            pl.BlockSpec((C2, 2), full2),
            pl.BlockSpec((C2, 1), full2),
            pl.BlockSpec((C2, 1), full2),
        ],
        out_specs=pl.BlockSpec((1, C2, T), lambda b, n: (b, 0, n)),
        out_shape=jax.ShapeDtypeStruct((B, C2, N), jnp.float32),
    )(x2, st2, g1, bt1)

    return out


# speedup vs baseline: 41.6202x; 2.5965x over previous
import functools

import jax
import jax.numpy as jnp
from jax.experimental import pallas as pl

_HI = jax.lax.Precision.HIGHEST


def _top3_weights(q, c, T, S):
    qn = jnp.sum(q * q, axis=1)
    cn = jnp.sum(c * c, axis=1)
    cross = jax.lax.dot_general(
        q.astype(jnp.bfloat16), c.astype(jnp.bfloat16),
        (((1,), (1,)), ((), ())),
        preferred_element_type=jnp.float32)
    d = (qn[:, None] - 2.0 * cross) + cn[None, :]
    inf = jnp.float32(jnp.inf)
    one = jnp.float32(1.0)
    zero = jnp.float32(0.0)
    bf = jnp.bfloat16
    m1 = jnp.min(d, axis=1)
    eq1 = d == m1[:, None]
    oh1 = jnp.where(eq1, one, zero).astype(bf)
    d1 = jnp.where(eq1, inf, d)
    m2 = jnp.min(d1, axis=1)
    eq2 = d1 == m2[:, None]
    oh2 = jnp.where(eq2, one, zero).astype(bf)
    d2 = jnp.where(eq2, inf, d1)
    m3 = jnp.min(d2, axis=1)
    oh3 = jnp.where(d2 == m3[:, None], one, zero).astype(bf)
    r1 = 1.0 / (m1 + 1e-8)
    r2 = 1.0 / (m2 + 1e-8)
    r3 = 1.0 / (m3 + 1e-8)
    norm = (r1 + r2) + r3
    return (oh1, oh2, oh3), (r1 / norm, r2 / norm, r3 / norm)


def _stage_a(xyz1_ref, xyz2_ref, p1_ref, p2_ref, w0_ref, b0_ref,
             x1_ref, st_ref, *, T, S):
    q = xyz1_ref[0]
    feats = [p1_ref[0]]
    for l in (1, 0):
        ohs, ws = _top3_weights(q, xyz2_ref[l, 0], T, S)
        pts = p2_ref[l, 0]
        hi = pts.astype(jnp.bfloat16)
        rem = pts - hi.astype(jnp.float32)
        mid = rem.astype(jnp.bfloat16)
        lo = (rem - mid.astype(jnp.float32)).astype(jnp.bfloat16)
        parts = jnp.concatenate([hi, mid, lo], axis=0)
        gs = []
        for oh in ohs:
            g3 = jax.lax.dot_general(
                parts, oh, (((1,), (1,)), ((), ())),
                preferred_element_type=jnp.float32)
            D = pts.shape[0]
            gs.append((g3[:D] + g3[D:2 * D]) + g3[2 * D:])
        interp_t = ((gs[0] * ws[0][None, :] + gs[1] * ws[1][None, :])
                    + gs[2] * ws[2][None, :])
        feats.append(interp_t)
    feat_t = jnp.concatenate(feats, axis=0)
    x1 = jax.lax.dot_general(
        w0_ref[...].astype(jnp.bfloat16), feat_t.astype(jnp.bfloat16),
        (((1,), (0,)), ((), ())), preferred_element_type=jnp.float32)
    x1 = x1 + b0_ref[...]
    x1_ref[0] = x1
    first = (pl.program_id(0) == 0) & (pl.program_id(1) == 0)

    @pl.when(first)
    def _():
        st_ref[...] = jnp.zeros_like(st_ref)

    s = jnp.sum(x1, axis=1, keepdims=True)
    sq = jnp.sum(x1 * x1, axis=1, keepdims=True)
    st_ref[...] += jnp.concatenate([s, sq], axis=1)


def _stage_b(x1_ref, st1_ref, g0_ref, bt0_ref, w1_ref, b1_ref,
             x2_ref, st_ref, *, count):
    st = st1_ref[...]
    mean = st[:, 0:1] * (1.0 / count)
    var = st[:, 1:2] * (1.0 / count) - mean * mean
    rstd = 1.0 / jnp.sqrt(var + 1e-5)
    h = (x1_ref[0] - mean) * (rstd * g0_ref[...]) + bt0_ref[...]
    h = jnp.maximum(h, 0.0)
    x2 = jax.lax.dot_general(
        w1_ref[...].astype(jnp.bfloat16), h.astype(jnp.bfloat16),
        (((1,), (0,)), ((), ())), preferred_element_type=jnp.float32)
    x2 = x2 + b1_ref[...]
    x2_ref[0] = x2
    first = (pl.program_id(0) == 0) & (pl.program_id(1) == 0)

    @pl.when(first)
    def _():
        st_ref[...] = jnp.zeros_like(st_ref)

    s = jnp.sum(x2, axis=1, keepdims=True)
    sq = jnp.sum(x2 * x2, axis=1, keepdims=True)
    st_ref[...] += jnp.concatenate([s, sq], axis=1)


def _stage_c(x2_ref, st2_ref, g1_ref, bt1_ref, out_ref, *, count):
    st = st2_ref[...]
    mean = st[:, 0:1] * (1.0 / count)
    var = st[:, 1:2] * (1.0 / count) - mean * mean
    rstd = 1.0 / jnp.sqrt(var + 1e-5)
    y = (x2_ref[0] - mean) * (rstd * g1_ref[...]) + bt1_ref[...]
    out_ref[0] = jnp.maximum(y, 0.0)


def kernel(xyz1, xyz2_list, points1, points2_list,
           conv_w0, conv_b0, gamma0, beta0,
           conv_w1, conv_b1, gamma1, beta1):
    B, _, N = xyz1.shape
    L, _, _, S = xyz2_list.shape
    D1 = points1.shape[1]
    D2 = points2_list.shape[2]
    C1 = conv_w0.shape[0]
    C2 = conv_w1.shape[0]
    T = 512 if N % 512 == 0 else N
    NT = N // T
    count = float(B * N)

    col = lambda v: v.reshape(-1, 1)
    b0, g0, bt0 = col(conv_b0), col(gamma0), col(beta0)
    b1, g1, bt1 = col(conv_b1), col(gamma1), col(beta1)
    xyz1_t = jnp.transpose(xyz1, (0, 2, 1))
    xyz2_t = jnp.transpose(xyz2_list, (0, 1, 3, 2))

    grid = (B, NT)
    full2 = lambda b, n: (0, 0)

    x1, st1 = pl.pallas_call(
        functools.partial(_stage_a, T=T, S=S),
        grid=grid,
        in_specs=[
            pl.BlockSpec((1, T, 3), lambda b, n: (b, n, 0)),
            pl.BlockSpec((L, 1, S, 3), lambda b, n: (0, b, 0, 0)),
            pl.BlockSpec((1, D1, T), lambda b, n: (b, 0, n)),
            pl.BlockSpec((L, 1, D2, S), lambda b, n: (0, b, 0, 0)),
            pl.BlockSpec((C1, D1 + L * D2), full2),
            pl.BlockSpec((C1, 1), full2),
        ],
        out_specs=[
            pl.BlockSpec((1, C1, T), lambda b, n: (b, 0, n)),
            pl.BlockSpec((C1, 2), full2),
        ],
        out_shape=[
            jax.ShapeDtypeStruct((B, C1, N), jnp.float32),
            jax.ShapeDtypeStruct((C1, 2), jnp.float32),
        ],
    )(xyz1_t, xyz2_t, points1, points2_list, conv_w0, b0)

    x2, st2 = pl.pallas_call(
        functools.partial(_stage_b, count=count),
        grid=grid,
        in_specs=[
            pl.BlockSpec((1, C1, T), lambda b, n: (b, 0, n)),
            pl.BlockSpec((C1, 2), full2),
            pl.BlockSpec((C1, 1), full2),
            pl.BlockSpec((C1, 1), full2),
            pl.BlockSpec((C2, C1), full2),
            pl.BlockSpec((C2, 1), full2),
        ],
        out_specs=[
            pl.BlockSpec((1, C2, T), lambda b, n: (b, 0, n)),
            pl.BlockSpec((C2, 2), full2),
        ],
        out_shape=[
            jax.ShapeDtypeStruct((B, C2, N), jnp.float32),
            jax.ShapeDtypeStruct((C2, 2), jnp.float32),
        ],
    )(x1, st1, g0, bt0, conv_w1, b1)

    out = pl.pallas_call(
        functools.partial(_stage_c, count=count),
        grid=grid,
        in_specs=[
            pl.BlockSpec((1, C2, T), lambda b, n: (b, 0, n)),
            pl.BlockSpec((C2, 2), full2),
            pl.BlockSpec((C2, 1), full2),
            pl.BlockSpec((C2, 1), full2),
        ],
        out_specs=pl.BlockSpec((1, C2, T), lambda b, n: (b, 0, n)),
        out_shape=jax.ShapeDtypeStruct((B, C2, N), jnp.float32),
    )(x2, st2, g1, bt1)

    return out
